# baseline (speedup 1.0000x reference)
import functools

import jax
import jax.numpy as jnp
from jax import lax
from jax.experimental import pallas as pl
from jax.experimental.pallas import tpu as pltpu, tpu_sc as plsc

B, C, T, H, W = 8, 64, 32, 56, 56
D = H * W
ROWS = B * C * T
NC, NS = 2, 16
NW = NC * NS
RPW = ROWS // NW
GPW = RPW // T
CHUNK = 16
NCHUNK = RPW // CHUNK
L = 16


def _mesh():
    return plsc.VectorSubcoreMesh(core_axis_name="c", subcore_axis_name="s")


@functools.partial(
    pl.kernel,
    out_type=jax.ShapeDtypeStruct((ROWS, D), jnp.float32),
    mesh=_mesh(),
    compiler_params=pltpu.CompilerParams(use_tc_tiling_on_sc=False),
    scratch_types=[
        pltpu.VMEM((T,), jnp.int32),
        pltpu.VMEM((RPW,), jnp.int32),
        pltpu.VMEM((CHUNK, D), jnp.float32),
        pltpu.VMEM((CHUNK, D), jnp.float32),
        pltpu.SemaphoreType.DMA,
        pltpu.SemaphoreType.DMA,
    ],
)
def _sc_shuffle(x_hbm, idx_hbm, out_hbm, idxs_v, rowidx_v, buf0, buf1, s0, s1):
    wid = lax.axis_index("s") * NC + lax.axis_index("c")
    base = wid * RPW

    pltpu.sync_copy(idx_hbm, idxs_v)
    for c in range(NCHUNK):
        g, h = divmod(c, 2)
        rowidx_v[pl.ds(c * L, L)] = idxs_v[pl.ds(h * L, L)] + (base + g * T)

    def gather(c, buf, sem):
        pltpu.async_copy(x_hbm.at[rowidx_v.at[pl.ds(c * CHUNK, CHUNK)]],
                         buf, sem)

    def gather_wait(c, buf, sem):
        pltpu.make_async_copy(x_hbm.at[rowidx_v.at[pl.ds(c * CHUNK, CHUNK)]],
                              buf, sem).wait()

    gather(0, buf0, s0)

    def body(i, _):
        c0 = 2 * i
        c1 = c0 + 1
        gather(c1, buf1, s1)
        gather_wait(c0, buf0, s0)
        pltpu.sync_copy(buf0, out_hbm.at[pl.ds(base + c0 * CHUNK, CHUNK)])

        @pl.when(i < NCHUNK // 2 - 1)
        def _():
            gather(c0 + 2, buf0, s0)

        gather_wait(c1, buf1, s1)
        pltpu.sync_copy(buf1, out_hbm.at[pl.ds(base + c1 * CHUNK, CHUNK)])
        return 0

    lax.fori_loop(0, NCHUNK // 2, body, 0)


def kernel(x, idxs):
    x2d = x.reshape(ROWS, D)
    out2d = _sc_shuffle(x2d, idxs.astype(jnp.int32))
    return out2d.reshape(B, C, T, H, W)


# speedup vs baseline: 5.3219x; 5.3219x over previous
import functools

import jax
import jax.numpy as jnp
from jax import lax
from jax.experimental import pallas as pl
from jax.experimental.pallas import tpu as pltpu, tpu_sc as plsc

B, C, T, H, W = 8, 64, 32, 56, 56
NC, NS = 2, 16
NW = NC * NS
SLABS = B * T
SPW = SLABS // NW
ROWS = SLABS * H
RCH = 8
CPS = H // RCH


@functools.partial(
    pl.kernel,
    out_type=jax.ShapeDtypeStruct((ROWS, W, C), jnp.float32),
    mesh=plsc.VectorSubcoreMesh(core_axis_name="c", subcore_axis_name="s"),
    scratch_types=[
        pltpu.VMEM((T + 16,), jnp.int32),
        pltpu.VMEM((RCH, W, C), jnp.float32),
        pltpu.VMEM((RCH, W, C), jnp.float32),
        pltpu.SemaphoreType.DMA,
        pltpu.SemaphoreType.DMA,
    ],
)
def _sc_shuffle(x_hbm, idx_hbm, out_hbm, idxs_v, buf0, buf1, s0, s1):
    wid = lax.axis_index("s") * NC + lax.axis_index("c")
    pltpu.sync_copy(idx_hbm, idxs_v.at[pl.ds(0, T)])

    bufs = (buf0, buf1)
    sems = (s0, s1)

    def src_row(c):
        s, k = divmod(c, CPS)
        d = wid * SPW + s
        b = lax.shift_right_logical(d, 5)
        j = lax.bitwise_and(d, T - 1)
        pj = idxs_v[pl.ds(j, 16)][0]
        return (b * T + pj) * H + k * RCH

    def dst_row(c):
        s, k = divmod(c, CPS)
        return (wid * SPW + s) * H + k * RCH

    def gather(c):
        pltpu.async_copy(x_hbm.at[pl.ds(src_row(c), RCH)],
                         bufs[c % 2], sems[c % 2])

    def gather_wait(c):
        pltpu.make_async_copy(x_hbm.at[pl.ds(src_row(c), RCH)],
                              bufs[c % 2], sems[c % 2]).wait()

    NCH = SPW * CPS
    gather(0)
    for c in range(NCH):
        if c + 1 < NCH:
            gather(c + 1)
        gather_wait(c)
        pltpu.sync_copy(bufs[c % 2], out_hbm.at[pl.ds(dst_row(c), RCH)])


def kernel(x, idxs):
    xt = jnp.transpose(x, (0, 2, 3, 4, 1))
    xr = xt.reshape(ROWS, W, C)
    out = _sc_shuffle(xr, idxs.astype(jnp.int32))
    out5 = out.reshape(B, T, H, W, C)
    return jnp.transpose(out5, (0, 4, 1, 2, 3))
